# baseline (speedup 1.0000x reference)
import functools

import jax
import jax.numpy as jnp
from jax import lax
from jax.experimental import pallas as pl
from jax.experimental.pallas import tpu as pltpu
from jax.experimental.pallas import tpu_sc as plsc

_B = 128
_N = 32768
_K = 3
_MARGIN_MIN = 0.1
_MARGIN_MAX = 0.3
_NEG = -50.0
_INIT = -3.0e38

_NW = 32
_ROWS_PER_W = _B // _NW
_BLK = 2048
_NBLK = _N // _BLK
_CHUNKS = _BLK // 16
_UNROLL = 4


def _bfly(x, op, lanes):
    for s in (8, 4, 2, 1):
        x = op(x, x.at[lanes ^ s].get(mode="promise_in_bounds"))
    return x


def _insert(m, t1, t2, t3):
    a = jnp.maximum(t1, m)
    b = jnp.minimum(t1, m)
    c = jnp.maximum(t2, b)
    d = jnp.minimum(t2, b)
    e = jnp.maximum(t3, d)
    return a, c, e


def _sc_body(inp_hbm, tgt_hbm, out_hbm,
             in0, in1, tg0, tg1, diag_v, acc_v,
             s_in0, s_in1, s_tg0, s_tg1, s_d):
    wid = lax.axis_index("c") * 16 + lax.axis_index("s")
    row0 = wid * _ROWS_PER_W
    in_bufs = (in0, in1)
    tg_bufs = (tg0, tg1)
    s_ins = (s_in0, s_in1)
    s_tgs = (s_tg0, s_tg1)

    def start(blk, p):
        c0 = blk * _BLK
        h_i = pltpu.async_copy(
            inp_hbm.at[pl.ds(row0, _ROWS_PER_W), pl.ds(c0, _BLK)],
            in_bufs[p], s_ins[p])
        h_t = pltpu.async_copy(
            tgt_hbm.at[pl.ds(row0, _ROWS_PER_W), pl.ds(c0, _BLK)],
            tg_bufs[p], s_tgs[p])
        return h_i, h_t

    h_d = pltpu.async_copy(
        inp_hbm.at[pl.ds(row0, _ROWS_PER_W), pl.ds(0, 128)], diag_v, s_d)

    handles = [None, None]
    handles[0] = start(0, 0)

    neg = jnp.full((16,), _NEG, jnp.float32)
    init = jnp.full((16,), _INIT, jnp.float32)
    stacks = tuple((init, init, init) for _ in range(_ROWS_PER_W))

    for blk in range(_NBLK):
        p = blk % 2
        if blk + 1 < _NBLK:
            handles[1 - p] = start(blk + 1, 1 - p)
        h_i, h_t = handles[p]
        h_i.wait()
        h_t.wait()
        ibuf = in_bufs[p]
        tbuf = tg_bufs[p]

        def body(i, carry, ibuf=ibuf, tbuf=tbuf):
            base = pl.multiple_of(i * (16 * _UNROLL), 16 * _UNROLL)
            cur = list(carry)
            for u in range(_UNROLL):
                for r in range(_ROWS_PER_W):
                    t1, t2, t3 = cur[r]
                    col = base + u * 16
                    v = ibuf[r, pl.ds(col, 16)]
                    tg = tbuf[r, pl.ds(col, 16)]
                    m = jnp.where(tg == 0.0, v, neg)
                    cur[r] = _insert(m, t1, t2, t3)
            return tuple(cur)

        stacks = lax.fori_loop(0, _CHUNKS // _UNROLL, body, stacks)

    h_d.wait()
    lanes = lax.iota(jnp.int32, 16)
    sixteen = jnp.full((16,), 16, jnp.int32)
    acc = jnp.zeros((16,), jnp.float32)
    for r in range(_ROWS_PER_W):
        t1, t2, t3 = stacks[r]
        dchunk = pl.multiple_of((row0 // 16) * 16, 16)
        drow = diag_v[r, pl.ds(dchunk, 16)]
        dlane = (row0 + r) % 16
        db = _bfly(jnp.where(lanes == dlane, drow, 0.0), jnp.add, lanes)
        for k in range(_K):
            mb = _bfly(t1, jnp.maximum, lanes)
            if k < _K - 1:
                lmin = _bfly(jnp.where(t1 == mb, lanes, sixteen),
                             jnp.minimum, lanes)
                sel = lanes == lmin
                t1 = jnp.where(sel, t2, t1)
                t2 = jnp.where(sel, t3, t2)
                t3 = jnp.where(sel, init, t3)
            marg = jnp.clip(jnp.abs(db - mb), _MARGIN_MIN, _MARGIN_MAX)
            acc = acc + jnp.maximum(mb - db + marg, 0.0)

    acc_v[...] = acc
    pltpu.sync_copy(acc_v, out_hbm.at[wid])


@jax.jit
def _sc_partials(inp, tgt):
    mesh = plsc.VectorSubcoreMesh(core_axis_name="c", subcore_axis_name="s")
    f = functools.partial(
        pl.kernel,
        out_type=jax.ShapeDtypeStruct((_NW, 16), jnp.float32),
        mesh=mesh,
        scratch_types=[
            pltpu.VMEM((_ROWS_PER_W, _BLK), jnp.float32),
            pltpu.VMEM((_ROWS_PER_W, _BLK), jnp.float32),
            pltpu.VMEM((_ROWS_PER_W, _BLK), jnp.float32),
            pltpu.VMEM((_ROWS_PER_W, _BLK), jnp.float32),
            pltpu.VMEM((_ROWS_PER_W, 128), jnp.float32),
            pltpu.VMEM((16,), jnp.float32),
            pltpu.SemaphoreType.DMA,
            pltpu.SemaphoreType.DMA,
            pltpu.SemaphoreType.DMA,
            pltpu.SemaphoreType.DMA,
            pltpu.SemaphoreType.DMA,
        ],
    )(_sc_body)
    return f(inp, tgt)


def kernel(input, target):
    partials = _sc_partials(input, target)
    return partials[:, 0].sum() / jnp.float32(_B * _K)


# speedup vs baseline: 1.0287x; 1.0287x over previous
import functools

import jax
import jax.numpy as jnp
from jax import lax
from jax.experimental import pallas as pl
from jax.experimental.pallas import tpu as pltpu
from jax.experimental.pallas import tpu_sc as plsc

_B = 128
_N = 32768
_K = 3
_MARGIN_MIN = 0.1
_MARGIN_MAX = 0.3
_NEG = -50.0
_INIT = -3.0e38

_NW = 32
_ROWS_PER_W = _B // _NW
_BLK = 2048
_NBLK = _N // _BLK
_CHUNKS = _BLK // 16
_UNROLL = 4


def _bfly(x, op, lanes):
    for s in (8, 4, 2, 1):
        x = op(x, x.at[lanes ^ s].get(mode="promise_in_bounds"))
    return x


def _insert(m, t1, t2, t3):
    a = jnp.maximum(t1, m)
    b = jnp.minimum(t1, m)
    c = jnp.maximum(t2, b)
    d = jnp.minimum(t2, b)
    e = jnp.maximum(t3, d)
    return a, c, e


def _sc_body(inp_hbm, tgt_hbm, out_hbm,
             in0, in1, tg0, tg1, diag_v, acc_v,
             s_in0, s_in1, s_tg0, s_tg1, s_d):
    wid = lax.axis_index("c") * 16 + lax.axis_index("s")
    row0 = wid * _ROWS_PER_W
    in_bufs = (in0, in1)
    tg_bufs = (tg0, tg1)
    s_ins = (s_in0, s_in1)
    s_tgs = (s_tg0, s_tg1)

    def start(blk, p):
        c0 = blk * _BLK
        h_i = pltpu.async_copy(
            inp_hbm.at[pl.ds(row0, _ROWS_PER_W), pl.ds(c0, _BLK)],
            in_bufs[p], s_ins[p])
        h_t = pltpu.async_copy(
            tgt_hbm.at[pl.ds(row0, _ROWS_PER_W), pl.ds(c0, _BLK)],
            tg_bufs[p], s_tgs[p])
        return h_i, h_t

    h_d = pltpu.async_copy(
        inp_hbm.at[pl.ds(row0, _ROWS_PER_W), pl.ds(0, 128)], diag_v, s_d)

    handles = [None, None]
    handles[0] = start(0, 0)

    neg = jnp.full((16,), _NEG, jnp.float32)
    init = jnp.full((16,), _INIT, jnp.float32)
    stacks = tuple((init, init, init) for _ in range(_ROWS_PER_W))

    for blk in range(_NBLK):
        p = blk % 2
        if blk + 1 < _NBLK:
            handles[1 - p] = start(blk + 1, 1 - p)
        h_i, h_t = handles[p]
        h_i.wait()
        h_t.wait()
        ibuf = in_bufs[p]
        tbuf = tg_bufs[p]

        @plsc.parallel_loop(0, _CHUNKS, 1, unroll=_UNROLL, carry=stacks)
        def stacks(i, carry, ibuf=ibuf, tbuf=tbuf):
            col = pl.multiple_of(i * 16, 16)
            cur = []
            for r in range(_ROWS_PER_W):
                t1, t2, t3 = carry[r]
                v = ibuf[r, pl.ds(col, 16)]
                tg = tbuf[r, pl.ds(col, 16)]
                m = jnp.where(tg == 0.0, v, neg)
                cur.append(_insert(m, t1, t2, t3))
            return tuple(cur)

    h_d.wait()
    lanes = lax.iota(jnp.int32, 16)
    sixteen = jnp.full((16,), 16, jnp.int32)
    acc = jnp.zeros((16,), jnp.float32)
    for r in range(_ROWS_PER_W):
        t1, t2, t3 = stacks[r]
        dchunk = pl.multiple_of((row0 // 16) * 16, 16)
        drow = diag_v[r, pl.ds(dchunk, 16)]
        dlane = (row0 + r) % 16
        db = _bfly(jnp.where(lanes == dlane, drow, 0.0), jnp.add, lanes)
        for k in range(_K):
            mb = _bfly(t1, jnp.maximum, lanes)
            if k < _K - 1:
                lmin = _bfly(jnp.where(t1 == mb, lanes, sixteen),
                             jnp.minimum, lanes)
                sel = lanes == lmin
                t1 = jnp.where(sel, t2, t1)
                t2 = jnp.where(sel, t3, t2)
                t3 = jnp.where(sel, init, t3)
            marg = jnp.clip(jnp.abs(db - mb), _MARGIN_MIN, _MARGIN_MAX)
            acc = acc + jnp.maximum(mb - db + marg, 0.0)

    acc_v[...] = acc
    pltpu.sync_copy(acc_v, out_hbm.at[wid])


@jax.jit
def _sc_partials(inp, tgt):
    mesh = plsc.VectorSubcoreMesh(core_axis_name="c", subcore_axis_name="s")
    f = functools.partial(
        pl.kernel,
        out_type=jax.ShapeDtypeStruct((_NW, 16), jnp.float32),
        mesh=mesh,
        scratch_types=[
            pltpu.VMEM((_ROWS_PER_W, _BLK), jnp.float32),
            pltpu.VMEM((_ROWS_PER_W, _BLK), jnp.float32),
            pltpu.VMEM((_ROWS_PER_W, _BLK), jnp.float32),
            pltpu.VMEM((_ROWS_PER_W, _BLK), jnp.float32),
            pltpu.VMEM((_ROWS_PER_W, 128), jnp.float32),
            pltpu.VMEM((16,), jnp.float32),
            pltpu.SemaphoreType.DMA,
            pltpu.SemaphoreType.DMA,
            pltpu.SemaphoreType.DMA,
            pltpu.SemaphoreType.DMA,
            pltpu.SemaphoreType.DMA,
        ],
    )(_sc_body)
    return f(inp, tgt)


def kernel(input, target):
    partials = _sc_partials(input, target)
    return partials[:, 0].sum() / jnp.float32(_B * _K)
